# baseline (speedup 1.0000x reference)
import functools

import jax
import jax.numpy as jnp
from jax import lax
from jax.experimental import pallas as pl
from jax.experimental.pallas import tpu as pltpu
from jax.experimental.pallas import tpu_sc as plsc

DIM = 64
B_ROWS = 16384 * 50
NUM_WORKERS = 32
B_PER_W = B_ROWS // NUM_WORKERS
CHUNK = 400
N_CHUNKS = B_PER_W // CHUNK
NBUF = 4
N_GROUPS = N_CHUNKS // NBUF


def _sc_gather(table, idx_flat):
    mesh = plsc.VectorSubcoreMesh(core_axis_name="c", subcore_axis_name="s")

    @functools.partial(
        pl.kernel,
        mesh=mesh,
        compiler_params=pltpu.CompilerParams(use_tc_tiling_on_sc=False),
        out_type=jax.ShapeDtypeStruct((B_ROWS, DIM), jnp.float32),
        scratch_types=[
            pltpu.VMEM((B_PER_W,), jnp.int32),
            pltpu.VMEM((NBUF, CHUNK, DIM), jnp.float32),
            pltpu.SemaphoreType.DMA,
            pltpu.SemaphoreType.DMA,
            pltpu.SemaphoreType.DMA,
            pltpu.SemaphoreType.DMA,
            pltpu.SemaphoreType.DMA,
            pltpu.SemaphoreType.DMA,
            pltpu.SemaphoreType.DMA,
            pltpu.SemaphoreType.DMA,
        ],
    )
    def k(table_hbm, idx_hbm, out_hbm, idx_v, rows_v,
          sg0, sg1, sg2, sg3, so0, so1, so2, so3):
        semg = (sg0, sg1, sg2, sg3)
        semo = (so0, so1, so2, so3)
        wid = lax.axis_index("s") * 2 + lax.axis_index("c")
        base = wid * B_PER_W

        pltpu.sync_copy(idx_hbm.at[pl.ds(base, B_PER_W)], idx_v)

        def start_gather(i, b):
            src = table_hbm.at[idx_v.at[pl.ds(i * CHUNK, CHUNK)]]
            pltpu.async_copy(src, rows_v.at[b], semg[b])

        def wait_gather(b):
            pltpu.make_async_copy(
                table_hbm.at[pl.ds(0, CHUNK)], rows_v.at[b], semg[b]).wait()

        def start_out(i, b):
            pltpu.async_copy(
                rows_v.at[b], out_hbm.at[pl.ds(base + i * CHUNK, CHUNK)],
                semo[b])

        def wait_out(i, b):
            pltpu.make_async_copy(
                rows_v.at[b], out_hbm.at[pl.ds(base + i * CHUNK, CHUNK)],
                semo[b]).wait()

        for b in range(NBUF):
            start_gather(b, b)

        def body(j, carry):
            i0 = j * NBUF
            for b in range(NBUF):
                wait_gather(b)
                start_out(i0 + b, b)
            for b in range(NBUF):
                wait_out(i0 + b, b)
                start_gather(i0 + NBUF + b, b)
            return carry

        lax.fori_loop(0, N_GROUPS - 1, body, 0)

        i0 = (N_GROUPS - 1) * NBUF
        for b in range(NBUF):
            wait_gather(b)
            start_out(i0 + b, b)
        for b in range(NBUF):
            wait_out(i0 + b, b)

    return k(table, idx_flat)


def kernel(bos_tensor, table):
    idx = bos_tensor.reshape(-1).astype(jnp.int32)
    out = _sc_gather(table, idx)
    return out.reshape(bos_tensor.shape[0], bos_tensor.shape[1], DIM)


# speedup vs baseline: 1.0012x; 1.0012x over previous
import functools

import jax
import jax.numpy as jnp
from jax import lax
from jax.experimental import pallas as pl
from jax.experimental.pallas import tpu as pltpu
from jax.experimental.pallas import tpu_sc as plsc

DIM = 64
SEQ = 50
BATCH = 16384
B_ROWS = BATCH * SEQ
NUM_WORKERS = 32
B_PER_W = B_ROWS // NUM_WORKERS
CHUNK = 400
BPC = CHUNK // SEQ
N_CHUNKS = B_PER_W // CHUNK
NBUF = 4
N_GROUPS = N_CHUNKS // NBUF


def _sc_gather(table, idx_flat):
    mesh = plsc.VectorSubcoreMesh(core_axis_name="c", subcore_axis_name="s")

    @functools.partial(
        pl.kernel,
        mesh=mesh,
        compiler_params=pltpu.CompilerParams(use_tc_tiling_on_sc=False),
        out_type=jax.ShapeDtypeStruct((BATCH, SEQ, DIM), jnp.float32),
        scratch_types=[
            pltpu.VMEM((B_PER_W,), jnp.int32),
            pltpu.VMEM((NBUF, CHUNK, DIM), jnp.float32),
            pltpu.SemaphoreType.DMA,
            pltpu.SemaphoreType.DMA,
            pltpu.SemaphoreType.DMA,
            pltpu.SemaphoreType.DMA,
            pltpu.SemaphoreType.DMA,
            pltpu.SemaphoreType.DMA,
            pltpu.SemaphoreType.DMA,
            pltpu.SemaphoreType.DMA,
        ],
    )
    def k(table_hbm, idx_hbm, out_hbm, idx_v, rows_v,
          sg0, sg1, sg2, sg3, so0, so1, so2, so3):
        semg = (sg0, sg1, sg2, sg3)
        semo = (so0, so1, so2, so3)
        wid = lax.axis_index("s") * 2 + lax.axis_index("c")
        base = wid * B_PER_W
        base_b = wid * (B_PER_W // SEQ)

        pltpu.sync_copy(idx_hbm.at[pl.ds(base, B_PER_W)], idx_v)

        def start_gather(i, n):
            src = table_hbm.at[idx_v.at[pl.ds(i * CHUNK, CHUNK)]]
            pltpu.async_copy(src, rows_v.at[n], semg[n])

        def wait_gather(n):
            pltpu.make_async_copy(
                table_hbm.at[pl.ds(0, CHUNK)], rows_v.at[n], semg[n]).wait()

        def start_out(i, n):
            b0 = base_b + i * BPC
            for j in range(BPC):
                pltpu.async_copy(
                    rows_v.at[n, pl.ds(j * SEQ, SEQ)], out_hbm.at[b0 + j],
                    semo[n])

        def wait_out(n):
            pltpu.make_async_copy(
                table_hbm.at[pl.ds(0, CHUNK)], rows_v.at[n], semo[n]).wait()

        for n in range(NBUF):
            start_gather(n, n)

        def body(j, carry):
            i0 = j * NBUF
            for n in range(NBUF):
                wait_gather(n)
                start_out(i0 + n, n)
            for n in range(NBUF):
                wait_out(n)
                start_gather(i0 + NBUF + n, n)
            return carry

        lax.fori_loop(0, N_GROUPS - 1, body, 0)

        i0 = (N_GROUPS - 1) * NBUF
        for n in range(NBUF):
            wait_gather(n)
            start_out(i0 + n, n)
        for n in range(NBUF):
            wait_out(n)

    return k(table, idx_flat)


def kernel(bos_tensor, table):
    idx = bos_tensor.reshape(-1).astype(jnp.int32)
    return _sc_gather(table, idx)


# speedup vs baseline: 1.0862x; 1.0849x over previous
import functools

import jax
import jax.numpy as jnp
from jax import lax
from jax.experimental import pallas as pl
from jax.experimental.pallas import tpu as pltpu
from jax.experimental.pallas import tpu_sc as plsc

DIM = 64
PAD_DIM = 128
SEQ = 50
BATCH = 16384
B_ROWS = BATCH * SEQ
NUM_WORKERS = 32
B_PER_W = B_ROWS // NUM_WORKERS
CHUNK = 128
N_CHUNKS = B_PER_W // CHUNK
NA = 4
NB = 4
BLOCK = 25
N_BLOCKS = N_CHUNKS // BLOCK
B_PER_BLOCK = BLOCK * CHUNK // SEQ

_PIECES = []
for _m in range(BLOCK):
    _s, _e = CHUNK * _m, CHUNK * (_m + 1)
    _p, _pos = [], _s
    while _pos < _e:
        _nxt = min(_e, (_pos // SEQ + 1) * SEQ)
        _p.append((_pos // SEQ, _pos % SEQ, _nxt - _pos))
        _pos = _nxt
    _PIECES.append(_p)


def _sc_gather(table_p, idx3):
    mesh = plsc.VectorSubcoreMesh(core_axis_name="c", subcore_axis_name="s")

    @functools.partial(
        pl.kernel,
        mesh=mesh,
        compiler_params=pltpu.CompilerParams(use_tc_tiling_on_sc=True),
        out_type=jax.ShapeDtypeStruct((BATCH, SEQ, DIM), jnp.float32),
        scratch_types=[
            pltpu.VMEM((N_CHUNKS, CHUNK), jnp.int32),
            pltpu.VMEM((NA, CHUNK, PAD_DIM), jnp.float32),
            pltpu.VMEM((NB, SEQ, DIM), jnp.float32),
            pltpu.SemaphoreType.DMA,
            pltpu.SemaphoreType.DMA,
            pltpu.SemaphoreType.DMA,
            pltpu.SemaphoreType.DMA,
            pltpu.SemaphoreType.DMA,
            pltpu.SemaphoreType.DMA,
            pltpu.SemaphoreType.DMA,
            pltpu.SemaphoreType.DMA,
        ],
    )
    def k(table_hbm, idx_hbm, out_hbm, idx_v, a_v, b_v,
          sg0, sg1, sg2, sg3, so0, so1, so2, so3):
        semg = (sg0, sg1, sg2, sg3)
        semo = (so0, so1, so2, so3)
        wid = lax.axis_index("s") * 2 + lax.axis_index("c")
        base_b = wid * (B_PER_W // SEQ)

        pltpu.sync_copy(idx_hbm.at[wid], idx_v)

        def start_gather(q, m, n):
            src = table_hbm.at[idx_v.at[q * BLOCK + m]]
            pltpu.async_copy(src, a_v.at[n], semg[n])

        def wait_gather(n):
            pltpu.make_async_copy(
                table_hbm.at[pl.ds(0, CHUNK)], a_v.at[n], semg[n]).wait()

        def start_slab_out(b, kb):
            pltpu.async_copy(b_v.at[kb], out_hbm.at[b], semo[kb])

        def wait_slab_out(b, kb):
            pltpu.make_async_copy(b_v.at[kb], out_hbm.at[b], semo[kb]).wait()

        def repack(n, rbase, kb, l0, nr):
            def body(i, carry):
                for c in range(DIM // 16):
                    b_v[kb, l0 + i, pl.ds(c * 16, 16)] = (
                        a_v[n, rbase + i, pl.ds(c * 16, 16)])
                return carry
            lax.fori_loop(0, nr, body, 0)

        def block(q, carry):
            b0 = base_b + q * B_PER_BLOCK
            for n in range(NA):
                start_gather(q, n, n)
            for m in range(BLOCK):
                n = m % NA
                wait_gather(n)
                rbase = 0
                for (db, l0, nr) in _PIECES[m]:
                    kb = db % NB
                    if l0 == 0 and db >= NB:
                        wait_slab_out(b0 + db - NB, kb)
                    repack(n, rbase, kb, l0, nr)
                    rbase += nr
                    if l0 + nr == SEQ:
                        start_slab_out(b0 + db, kb)
                if m + NA < BLOCK:
                    start_gather(q, m + NA, n)
            for kb in range(NB):
                wait_slab_out(b0 + B_PER_BLOCK - NB + kb, kb)
            return carry

        lax.fori_loop(0, N_BLOCKS, block, 0)

    return k(table_p, idx3)


def kernel(bos_tensor, table):
    idx = bos_tensor.reshape(-1).astype(jnp.int32)
    idx3 = idx.reshape(NUM_WORKERS, N_CHUNKS, CHUNK)
    table_p = jnp.pad(table, ((0, 0), (0, PAD_DIM - DIM)))
    return _sc_gather(table_p, idx3)
